# baseline (speedup 1.0000x reference)
import functools

import jax
import jax.numpy as jnp
from jax import lax
from jax.experimental import pallas as pl
from jax.experimental.pallas import tpu as pltpu
from jax.experimental.pallas import tpu_sc as plsc

_BATCH = 16384
_DIM = 64
_LANES = 16
_NUM_WORKERS = 32
_BPW = _BATCH // _NUM_WORKERS
_CHUNK = 128
_NCHUNK = _BPW // _CHUNK
_GROUPS = _CHUNK // _LANES


def _vec_sqrt(x):
    i = lax.bitcast_convert_type(x, jnp.int32)
    i = jnp.int32(0x1FBD1DF5) + lax.shift_right_logical(i, 1)
    y = lax.bitcast_convert_type(i, jnp.float32)
    for _ in range(3):
        y = 0.5 * (y + x / y)
    return y


def _tec_body(head, relation, tail, ent2, rel2, out,
              hidx, ridx, tidx, hsh, rsh, tsh, hbuf, rbuf, tbuf, outv, sem):
    wid = lax.axis_index("s") * 2 + lax.axis_index("c")
    base = wid * _BPW

    for j in range(_NCHUNK):
        src = pl.ds(base + j * _CHUNK, _CHUNK)
        dst = pl.ds(j * _CHUNK, _CHUNK)
        pltpu.sync_copy(head.at[src], hidx.at[dst])
        pltpu.sync_copy(relation.at[src], ridx.at[dst])
        pltpu.sync_copy(tail.at[src], tidx.at[dst])

    def shift_body(v, carry):
        sl = pl.ds(v * _LANES, _LANES)
        hsh[sl] = lax.shift_right_logical(hidx[sl], 1)
        rsh[sl] = lax.shift_right_logical(ridx[sl], 1)
        tsh[sl] = lax.shift_right_logical(tidx[sl], 1)
        return carry

    lax.fori_loop(0, _BPW // _LANES, shift_body, 0)

    row_iota = lax.iota(jnp.int32, _LANES)

    for j in range(_NCHUNK):
        csl = pl.ds(j * _CHUNK, _CHUNK)
        copies = [
            pltpu.async_copy(ent2.at[hsh.at[csl]], hbuf, sem),
            pltpu.async_copy(rel2.at[rsh.at[csl]], rbuf, sem),
            pltpu.async_copy(ent2.at[tsh.at[csl]], tbuf, sem),
        ]
        for cp in copies:
            cp.wait()

        def group_body(g, carry, j=j):
            rows = g * _LANES + row_iota
            gsl = pl.ds(j * _CHUNK + g * _LANES, _LANES)
            hp = lax.shift_left(hidx[gsl] & 1, 6)
            rp = lax.shift_left(ridx[gsl] & 1, 6)
            tp = lax.shift_left(tidx[gsl] & 1, 6)
            acc0 = jnp.zeros((_LANES,), jnp.float32)
            acc1 = jnp.zeros((_LANES,), jnp.float32)
            for d in range(_DIM):
                hv = plsc.load_gather(hbuf, [rows, hp])
                rv = plsc.load_gather(rbuf, [rows, rp])
                tv = plsc.load_gather(tbuf, [rows, tp])
                s = hv + rv - tv
                if d % 2 == 0:
                    acc0 = acc0 + s * s
                else:
                    acc1 = acc1 + s * s
                if d != _DIM - 1:
                    hp = hp + 1
                    rp = rp + 1
                    tp = tp + 1
            outv[pl.ds(j * _CHUNK + g * _LANES, _LANES)] = _vec_sqrt(acc0 + acc1)
            return carry

        lax.fori_loop(0, _GROUPS, group_body, 0)

    pltpu.sync_copy(outv, out.at[pl.ds(base, _BPW)])


@functools.partial(
    pl.kernel,
    out_type=jax.ShapeDtypeStruct((_BATCH,), jnp.float32),
    mesh=plsc.VectorSubcoreMesh(core_axis_name="c", subcore_axis_name="s"),
    compiler_params=pltpu.CompilerParams(needs_layout_passes=False),
    scratch_types=[
        pltpu.VMEM((_BPW,), jnp.int32),
        pltpu.VMEM((_BPW,), jnp.int32),
        pltpu.VMEM((_BPW,), jnp.int32),
        pltpu.VMEM((_BPW,), jnp.int32),
        pltpu.VMEM((_BPW,), jnp.int32),
        pltpu.VMEM((_BPW,), jnp.int32),
        pltpu.VMEM((_CHUNK, 2 * _DIM), jnp.float32),
        pltpu.VMEM((_CHUNK, 2 * _DIM), jnp.float32),
        pltpu.VMEM((_CHUNK, 2 * _DIM), jnp.float32),
        pltpu.VMEM((_BPW,), jnp.float32),
        pltpu.SemaphoreType.DMA,
    ],
)
def _transe_sc(*args):
    _tec_body(*args)


def kernel(head, relation, tail, entity_table, relation_table):
    ent2 = entity_table.reshape(entity_table.shape[0] // 2, 2 * _DIM)
    rel2 = relation_table.reshape(relation_table.shape[0] // 2, 2 * _DIM)
    return _transe_sc(head, relation, tail, ent2, rel2)


# speedup vs baseline: 1.7306x; 1.7306x over previous
import functools

import jax
import jax.numpy as jnp
from jax import lax
from jax.experimental import pallas as pl
from jax.experimental.pallas import tpu as pltpu
from jax.experimental.pallas import tpu_sc as plsc

_BATCH = 16384
_DIM = 64
_LANES = 16
_NUM_WORKERS = 32
_BPW = _BATCH // _NUM_WORKERS
_NCH = _BPW // _LANES


def _vec_sqrt(x):
    i = lax.bitcast_convert_type(x, jnp.int32)
    i = jnp.int32(0x1FBD1DF5) + lax.shift_right_logical(i, 1)
    y = lax.bitcast_convert_type(i, jnp.float32)
    for _ in range(3):
        y = 0.5 * (y + x / y)
    return y


def _tec_body(head, relation, tail, ent, rel, out,
              hidx, ridx, tidx, hbuf, rbuf, tbuf, outv, sem):
    wid = lax.axis_index("s") * 2 + lax.axis_index("c")
    base = wid * _BPW

    src = pl.ds(base, _BPW)
    pltpu.sync_copy(head.at[src], hidx)
    pltpu.sync_copy(relation.at[src], ridx)
    pltpu.sync_copy(tail.at[src], tidx)

    row_iota = lax.iota(jnp.int32, _LANES)

    def chunk_body(g, carry):
        sl = pl.ds(g * _LANES, _LANES)
        hv = hidx[sl]
        rv = ridx[sl]
        tv = tidx[sl]
        copies = []
        for j in range(_LANES):
            copies.append(pltpu.async_copy(
                ent.at[pl.ds(hv[j], 1), :], hbuf.at[pl.ds(j, 1), :], sem))
            copies.append(pltpu.async_copy(
                rel.at[pl.ds(rv[j], 1), :], rbuf.at[pl.ds(j, 1), :], sem))
            copies.append(pltpu.async_copy(
                ent.at[pl.ds(tv[j], 1), :], tbuf.at[pl.ds(j, 1), :], sem))
        for cp in copies:
            cp.wait()

        res = jnp.zeros((_LANES,), jnp.float32)
        for e in range(_LANES):
            acc = jnp.zeros((_LANES,), jnp.float32)
            for c in range(_DIM // _LANES):
                csl = pl.ds(c * _LANES, _LANES)
                s = hbuf[e, csl] + rbuf[e, csl] - tbuf[e, csl]
                acc = acc + s * s
            res = jnp.where(row_iota == e, jnp.sum(acc), res)
        outv[sl] = _vec_sqrt(res)
        return carry

    lax.fori_loop(0, _NCH, chunk_body, 0)

    pltpu.sync_copy(outv, out.at[pl.ds(base, _BPW)])


@functools.partial(
    pl.kernel,
    out_type=jax.ShapeDtypeStruct((_BATCH,), jnp.float32),
    mesh=plsc.VectorSubcoreMesh(core_axis_name="c", subcore_axis_name="s"),
    compiler_params=pltpu.CompilerParams(needs_layout_passes=False),
    scratch_types=[
        pltpu.VMEM((_BPW,), jnp.int32),
        pltpu.VMEM((_BPW,), jnp.int32),
        pltpu.VMEM((_BPW,), jnp.int32),
        pltpu.VMEM((_LANES, _DIM), jnp.float32),
        pltpu.VMEM((_LANES, _DIM), jnp.float32),
        pltpu.VMEM((_LANES, _DIM), jnp.float32),
        pltpu.VMEM((_BPW,), jnp.float32),
        pltpu.SemaphoreType.DMA,
    ],
)
def _transe_sc(*args):
    _tec_body(*args)


def kernel(head, relation, tail, entity_table, relation_table):
    return _transe_sc(head, relation, tail, entity_table, relation_table)
